# baseline (speedup 1.0000x reference)
import functools

import jax
import jax.numpy as jnp
from jax import lax
from jax.experimental import pallas as pl
from jax.experimental.pallas import tpu as pltpu
from jax.experimental.pallas import tpu_sc as plsc

_N = 77
_D = 768
_RPW = 3
_FULL_W = _N // _RPW
_TAIL_BASE = _FULL_W * _RPW
_TAIL_ROWS = _N - _TAIL_BASE


def _matmul_body(x_ref, wv_ref, wo_ref, bo_ref, wn_ref, bn_ref, p_ref):
    x = x_ref[...]
    t = jnp.dot(x, wv_ref[...], preferred_element_type=jnp.float32)
    t = jnp.dot(t, wo_ref[...], preferred_element_type=jnp.float32) + bo_ref[...]
    p_ref[...] = jnp.dot(t, wn_ref[...], preferred_element_type=jnp.float32) + bn_ref[...]


def _sc_scatter_body(tok_hbm, ph_hbm, p_hbm, emb_hbm, out_hbm, tok_v, ph_v, ind_v):
    c = lax.axis_index("c")
    s = lax.axis_index("s")
    wid = s * 2 + c
    base = wid * _RPW
    off = pl.multiple_of(base * _D, 8)

    @pl.when(wid < _FULL_W)
    def _():
        pltpu.sync_copy(emb_hbm.at[pl.ds(off, _RPW * _D)],
                        out_hbm.at[pl.ds(off, _RPW * _D)])

    @pl.when(wid == _FULL_W)
    def _():
        pltpu.sync_copy(emb_hbm.at[pl.ds(_TAIL_BASE * _D, _TAIL_ROWS * _D)],
                        out_hbm.at[pl.ds(_TAIL_BASE * _D, _TAIL_ROWS * _D)])

    @pl.when(wid <= _FULL_W)
    def _():
        pltpu.sync_copy(tok_hbm, tok_v)
        pltpu.sync_copy(ph_hbm, ph_v)
        phv = ph_v[...]
        for ch in range(5):
            tokv = tok_v[pl.ds(ch * 16, 16)]
            ind_v[pl.ds(ch * 16, 16)] = 1 - jnp.minimum(jnp.abs(tokv - phv), 1)
        for k in range(_RPW):
            r = base + k
            hit = ind_v[pl.ds(r, 16)][0]

            @pl.when((hit > 0) & (r < _N))
            def _():
                roff = pl.multiple_of(r * _D, 8)
                pltpu.sync_copy(p_hbm, out_hbm.at[pl.ds(roff, _D)])


def kernel(tokenized_text, embedded_text, image_embeds, placeholder_token,
           Wq1, Wk1, Wv1, Wo1, bo1, Wq2, Wk2, Wv2, Wo2, bo2, Wn, bn):
    b, n = tokenized_text.shape
    d = embedded_text.shape[-1]
    x = image_embeds.reshape(1, d)

    p = pl.pallas_call(
        _matmul_body,
        out_shape=jax.ShapeDtypeStruct((1, d), jnp.float32),
        in_specs=[pl.BlockSpec(memory_space=pltpu.VMEM)] * 6,
        out_specs=pl.BlockSpec(memory_space=pltpu.VMEM),
    )(x, Wv2, Wo2, bo2.reshape(1, d), Wn, bn.reshape(1, d))

    tok80 = jnp.zeros((80,), jnp.int32).at[:n].set(tokenized_text[0])
    ph16 = jnp.broadcast_to(placeholder_token, (16,))
    emb = embedded_text.reshape(n * d)

    scatter = functools.partial(
        pl.kernel,
        mesh=plsc.VectorSubcoreMesh(core_axis_name="c", subcore_axis_name="s"),
        out_type=jax.ShapeDtypeStruct((n * d,), jnp.float32),
        scratch_types=[
            pltpu.VMEM((80,), jnp.int32),
            pltpu.VMEM((16,), jnp.int32),
            pltpu.VMEM((96,), jnp.int32),
        ],
    )(_sc_scatter_body)
    out = scatter(tok80, ph16, p.reshape(d), emb)
    return out.reshape(b, n, d)


# speedup vs baseline: 2.9242x; 2.9242x over previous
import jax
import jax.numpy as jnp
from jax.experimental import pallas as pl
from jax.experimental.pallas import tpu as pltpu

_STEPS = 3


def _fused_body(ph_ref, tok_ref, emb_ref, x_ref, wv_ref, wo_ref, bo_ref,
                wn_ref, bn_ref, out_ref, t_s, p_s):
    j = pl.program_id(0)

    @pl.when(j == 0)
    def _():
        t_s[...] = jnp.dot(x_ref[...], wv_ref[...],
                           preferred_element_type=jnp.float32)

    u = jnp.dot(t_s[...], wo_ref[...],
                preferred_element_type=jnp.float32) + bo_ref[...]
    pu = jnp.dot(u, wn_ref[...], preferred_element_type=jnp.float32)

    @pl.when(j == 0)
    def _():
        p_s[...] = pu

    @pl.when(j > 0)
    def _():
        p_s[...] += pu

    @pl.when(j == _STEPS - 1)
    def _():
        p = p_s[...] + bn_ref[...]
        mask = tok_ref[...] == ph_ref[0]
        out_ref[...] = jnp.where(mask, p, emb_ref[...])


def kernel(tokenized_text, embedded_text, image_embeds, placeholder_token,
           Wq1, Wk1, Wv1, Wo1, bo1, Wq2, Wk2, Wv2, Wo2, bo2, Wn, bn):
    b, n = tokenized_text.shape
    d = embedded_text.shape[-1]
    inner = Wv2.shape[-1]
    ch = d // _STEPS
    tok = tokenized_text.reshape(n, 1)
    emb = embedded_text.reshape(n, d)
    x = image_embeds.reshape(1, d)
    ph = placeholder_token.reshape(1)
    out = pl.pallas_call(
        _fused_body,
        grid=(_STEPS,),
        out_shape=jax.ShapeDtypeStruct((n, d), jnp.float32),
        in_specs=[
            pl.BlockSpec(memory_space=pltpu.SMEM),
            pl.BlockSpec((n, 1), lambda j: (0, 0)),
            pl.BlockSpec((n, d), lambda j: (0, 0)),
            pl.BlockSpec((1, d), lambda j: (0, 0)),
            pl.BlockSpec((d, inner), lambda j: (0, 0)),
            pl.BlockSpec((inner, ch), lambda j: (0, j)),
            pl.BlockSpec((1, ch), lambda j: (0, j)),
            pl.BlockSpec((ch, d), lambda j: (j, 0)),
            pl.BlockSpec((1, d), lambda j: (0, 0)),
        ],
        out_specs=pl.BlockSpec((n, d), lambda j: (0, 0)),
        scratch_shapes=[
            pltpu.VMEM((1, inner), jnp.float32),
            pltpu.VMEM((1, d), jnp.float32),
        ],
    )(ph, tok, emb, x, Wv2, Wo2, bo2.reshape(1, d), Wn, bn.reshape(1, d))
    return out.reshape(b, n, d)


# speedup vs baseline: 3.1895x; 1.0907x over previous
import jax
import jax.numpy as jnp
from jax.experimental import pallas as pl
from jax.experimental.pallas import tpu as pltpu


def _fused_body(ph_ref, tok_ref, emb_ref, x_ref, wv_ref, wo_ref, bo_ref,
                wn_ref, bn_ref, out_ref):
    x = x_ref[...]
    t = jnp.dot(x, wv_ref[...], preferred_element_type=jnp.float32)
    t = jnp.dot(t, wo_ref[...], preferred_element_type=jnp.float32) + bo_ref[...]
    p = jnp.dot(t, wn_ref[...], preferred_element_type=jnp.float32) + bn_ref[...]
    mask = tok_ref[...] == ph_ref[0]
    out_ref[...] = jnp.where(mask, p, emb_ref[...])


def kernel(tokenized_text, embedded_text, image_embeds, placeholder_token,
           Wq1, Wk1, Wv1, Wo1, bo1, Wq2, Wk2, Wv2, Wo2, bo2, Wn, bn):
    b, n = tokenized_text.shape
    d = embedded_text.shape[-1]
    tok = tokenized_text.reshape(n, 1)
    emb = embedded_text.reshape(n, d)
    x = image_embeds.reshape(1, d)
    ph = placeholder_token.reshape(1)
    out = pl.pallas_call(
        _fused_body,
        out_shape=jax.ShapeDtypeStruct((n, d), jnp.float32),
        in_specs=[
            pl.BlockSpec(memory_space=pltpu.SMEM),
            pl.BlockSpec(memory_space=pltpu.VMEM),
            pl.BlockSpec(memory_space=pltpu.VMEM),
            pl.BlockSpec(memory_space=pltpu.VMEM),
            pl.BlockSpec(memory_space=pltpu.VMEM),
            pl.BlockSpec(memory_space=pltpu.VMEM),
            pl.BlockSpec(memory_space=pltpu.VMEM),
            pl.BlockSpec(memory_space=pltpu.VMEM),
            pl.BlockSpec(memory_space=pltpu.VMEM),
        ],
        out_specs=pl.BlockSpec(memory_space=pltpu.VMEM),
    )(ph, tok, emb, x, Wv2, Wo2, bo2.reshape(1, d), Wn, bn.reshape(1, d))
    return out.reshape(b, n, d)


# speedup vs baseline: 3.2724x; 1.0260x over previous
import jax
import jax.numpy as jnp
from jax import lax
from jax.experimental import pallas as pl
from jax.experimental.pallas import tpu as pltpu


def _fused_body(ph_ref, tok_ref, emb_ref, x_ref, wv_ref, wo_ref, bo_ref,
                wn_ref, bn_ref, out_ref):
    n, d = out_ref.shape
    np_ = tok_ref.shape[1]
    x = x_ref[...]
    t = jnp.dot(x, wv_ref[...], preferred_element_type=jnp.float32)
    t = jnp.dot(t, wo_ref[...], preferred_element_type=jnp.float32) + bo_ref[...]
    p = jnp.dot(t, wn_ref[...], preferred_element_type=jnp.float32) + bn_ref[...]
    m = (tok_ref[...] == ph_ref[0]).astype(jnp.float32)
    rows = lax.broadcasted_iota(jnp.int32, (n, np_), 0)
    cols = lax.broadcasted_iota(jnp.int32, (n, np_), 1)
    eye = (rows == cols).astype(jnp.float32)
    maskcol = lax.dot_general(eye, m, (((1,), (1,)), ((), ())),
                              preferred_element_type=jnp.float32)
    out_ref[...] = jnp.where(maskcol > 0.5, p, emb_ref[...])


def kernel(tokenized_text, embedded_text, image_embeds, placeholder_token,
           Wq1, Wk1, Wv1, Wo1, bo1, Wq2, Wk2, Wv2, Wo2, bo2, Wn, bn):
    b, n = tokenized_text.shape
    d = embedded_text.shape[-1]
    npad = (n + 7) // 8 * 8
    tok = jnp.pad(tokenized_text.reshape(1, n), ((0, 0), (0, npad - n)))
    emb = embedded_text.reshape(n, d)
    x = image_embeds.reshape(1, d)
    ph = placeholder_token.reshape(1)
    out = pl.pallas_call(
        _fused_body,
        out_shape=jax.ShapeDtypeStruct((n, d), jnp.float32),
        in_specs=[
            pl.BlockSpec(memory_space=pltpu.SMEM),
            pl.BlockSpec(memory_space=pltpu.VMEM),
            pl.BlockSpec(memory_space=pltpu.VMEM),
            pl.BlockSpec(memory_space=pltpu.VMEM),
            pl.BlockSpec(memory_space=pltpu.VMEM),
            pl.BlockSpec(memory_space=pltpu.VMEM),
            pl.BlockSpec(memory_space=pltpu.VMEM),
            pl.BlockSpec(memory_space=pltpu.VMEM),
            pl.BlockSpec(memory_space=pltpu.VMEM),
        ],
        out_specs=pl.BlockSpec(memory_space=pltpu.VMEM),
    )(ph, tok, emb, x, Wv2, Wo2, bo2.reshape(1, d), Wn, bn.reshape(1, d))
    return out.reshape(b, n, d)
